# Initial kernel scaffold; baseline (speedup 1.0000x reference)
import jax
import jax.numpy as jnp
from jax.experimental import pallas as pl


def kernel(x, weights):
    raise NotImplementedError("write your pallas kernel here")


# speedup vs baseline: 5.7560x; 5.7560x over previous
import functools

import jax
import jax.numpy as jnp
from jax import lax
from jax.experimental import pallas as pl
from jax.experimental.pallas import tpu as pltpu
from jax.experimental.pallas import tpu_sc as plsc

_K = 4
_L = 16
_NC = 2
_NS = 16
_NW = _NC * _NS
_NEG = -1.0e30


def _insert_top4(ms, v):
    m0, m1, m2, m3 = ms
    hi0 = jnp.maximum(m0, v)
    lo0 = jnp.minimum(m0, v)
    hi1 = jnp.maximum(m1, lo0)
    lo1 = jnp.minimum(m1, lo0)
    hi2 = jnp.maximum(m2, lo1)
    lo2 = jnp.minimum(m2, lo1)
    hi3 = jnp.maximum(m3, lo2)
    return (hi0, hi1, hi2, hi3)


def _merge_weighted(ms, wvec):
    lanes = lax.iota(jnp.int32, _L)
    negvec = jnp.full((_L,), _NEG, jnp.float32)
    val = jnp.zeros((_L,), jnp.float32)
    p = jnp.int32(0)
    ms = list(ms)
    for _ in range(_K):
        big = jnp.maximum(jnp.maximum(ms[0], ms[1]), jnp.maximum(ms[2], ms[3]))
        g = jnp.max(big)
        gs = jnp.full((_L,), g)
        eqs = [m == gs for m in ms]
        cnt_v = sum(jnp.where(eq, jnp.int32(1), jnp.int32(0)) for eq in eqs)
        pn = p + jnp.sum(cnt_v)
        sel = (lanes >= p) & (lanes < pn)
        val = jnp.where(sel, gs, val)
        ms = [jnp.where(eq, negvec, m) for eq, m in zip(eqs, ms)]
        p = pn
    return jnp.sum(val * wvec)


def _row_result(load, row_len, wvec):
    def ins(j, ms):
        return _insert_top4(ms, load(j))

    negvec = jnp.full((_L,), _NEG, jnp.float32)
    ms = lax.fori_loop(0, row_len // _L, ins, (negvec, negvec, negvec, negvec))
    return _merge_weighted(ms, wvec)


def _make_pool(n_rows, row_len, chunk_rows):
    rows_per_w = n_rows // _NW
    n_chunks = rows_per_w // chunk_rows
    chunk_elems = chunk_rows * row_len
    mesh = plsc.VectorSubcoreMesh(core_axis_name="c", subcore_axis_name="s")

    def body(x_hbm, w_hbm, out_hbm, wv, buf0, buf1, outv, sem0, sem1):
        cid = lax.axis_index("c")
        sid = lax.axis_index("s")
        wid = sid * _NC + cid
        elem_base = wid * rows_per_w * row_len

        pltpu.sync_copy(w_hbm, wv)
        wvec = wv[...]

        def start(ci, buf, sem):
            return pltpu.async_copy(
                x_hbm.at[pl.ds(elem_base + ci * chunk_elems, chunk_elems)],
                buf, sem)

        start(0, buf0, sem0)
        start(1, buf1, sem1)

        def process(ci, buf, sem):
            pltpu.make_async_copy(
                x_hbm.at[pl.ds(elem_base, chunk_elems)], buf, sem).wait()

            def row_body(r, results):
                def load(j):
                    return buf[pl.ds(r * row_len + j * _L, _L)]
                res = _row_result(load, row_len, wvec)
                lanes = lax.iota(jnp.int32, _L)
                return jnp.where(lanes == r, jnp.full((_L,), res), results)

            results = lax.fori_loop(
                0, chunk_rows, row_body, jnp.zeros((_L,), jnp.float32))
            outv[pl.ds(ci * chunk_rows, chunk_rows)] = results

            @pl.when(ci + 2 < n_chunks)
            def _():
                start(ci + 2, buf, sem)

        def pair_body(p, carry):
            process(2 * p, buf0, sem0)
            process(2 * p + 1, buf1, sem1)
            return carry

        lax.fori_loop(0, n_chunks // 2, pair_body, jnp.int32(0))

        pltpu.sync_copy(outv, out_hbm.at[pl.ds(wid * rows_per_w, rows_per_w)])

    return pl.kernel(
        body,
        out_type=jax.ShapeDtypeStruct((n_rows,), jnp.float32),
        mesh=mesh,
        compiler_params=pltpu.CompilerParams(needs_layout_passes=False),
        scratch_types=[
            pltpu.VMEM((_L,), jnp.float32),
            pltpu.VMEM((chunk_elems,), jnp.float32),
            pltpu.VMEM((chunk_elems,), jnp.float32),
            pltpu.VMEM((rows_per_w,), jnp.float32),
            pltpu.SemaphoreType.DMA,
            pltpu.SemaphoreType.DMA,
        ],
    )


def kernel(x, weights):
    b, c, h, w = x.shape
    n_rows = b * c
    row_len = h * w
    assert row_len % _L == 0 and n_rows % _NW == 0
    chunk_rows = _L
    xf = x.reshape(n_rows * row_len)
    wvec = jnp.zeros((_L,), jnp.float32).at[:_K].set(
        weights.reshape(-1).astype(jnp.float32) / _K)
    pool = _make_pool(n_rows, row_len, chunk_rows)
    out = pool(xf, wvec)
    return out.reshape(b, c, 1, 1)


# speedup vs baseline: 8.2512x; 1.4335x over previous
import numpy as np

import jax
import jax.numpy as jnp
from jax import lax
from jax.experimental import pallas as pl
from jax.experimental.pallas import tpu as pltpu
from jax.experimental.pallas import tpu_sc as plsc

_K = 4
_L = 16
_NC = 2
_NS = 16
_NW = _NC * _NS
_NEG = -1.0e30


def _sort4(a, b, c, d):
    a, b = jnp.maximum(a, b), jnp.minimum(a, b)
    c, d = jnp.maximum(c, d), jnp.minimum(c, d)
    a, c = jnp.maximum(a, c), jnp.minimum(a, c)
    b, d = jnp.maximum(b, d), jnp.minimum(b, d)
    b, c = jnp.maximum(b, c), jnp.minimum(b, c)
    return a, b, c, d


def _merge44(ms, bs):
    a0, a1, a2, a3 = ms
    b0, b1, b2, b3 = bs
    m00 = jnp.minimum(a0, b0)
    m01 = jnp.minimum(a0, b1)
    m10 = jnp.minimum(a1, b0)
    m02 = jnp.minimum(a0, b2)
    m11 = jnp.minimum(a1, b1)
    m20 = jnp.minimum(a2, b0)
    c0 = jnp.maximum(a0, b0)
    c1 = jnp.maximum(m00, jnp.maximum(a1, b1))
    c2 = jnp.maximum(jnp.maximum(b2, a2), jnp.maximum(m01, m10))
    c3 = jnp.maximum(jnp.maximum(b3, a3),
                     jnp.maximum(m02, jnp.maximum(m11, m20)))
    return c0, c1, c2, c3


def _absorb4(ms, vs):
    return _merge44(ms, _sort4(*vs))


def _make_pool(n_rows, row_len, chunk_rows):
    assert chunk_rows == _L
    rows_per_w = n_rows // _NW
    n_groups = rows_per_w // chunk_rows
    group_elems = chunk_rows * row_len
    mesh = plsc.VectorSubcoreMesh(core_axis_name="c", subcore_axis_name="s")

    main_steps = row_len - _L + 1
    unroll = _L
    n_iters = main_steps // unroll
    tail_start = n_iters * unroll
    slice_len = (_L - 1) * row_len + _L - 1 + unroll

    def body(x_hbm, w_hbm, out_hbm, wv, buf0, buf1, outv, sem0, sem1):
        cid = lax.axis_index("c")
        sid = lax.axis_index("s")
        wid = sid * _NC + cid
        elem_base = wid * rows_per_w * row_len

        pltpu.sync_copy(w_hbm, wv)
        wr = [wv[t, :] for t in range(_K)]

        lanes = lax.iota(jnp.int32, _L)
        diag = lanes * row_len + lanes
        idx_main = [diag + t for t in range(unroll)]
        idx_tail = [
            jnp.where(lanes + j >= row_len, diag + (j - row_len), diag + j)
            for j in range(tail_start, row_len)
        ]

        def start(gi, buf, sem):
            return pltpu.async_copy(
                x_hbm.at[pl.ds(elem_base + gi * group_elems, group_elems)],
                buf, sem)

        start(0, buf0, sem0)
        start(1, buf1, sem1)

        def process(gi, buf, sem):
            pltpu.make_async_copy(
                x_hbm.at[pl.ds(elem_base, group_elems)], buf, sem).wait()

            negv = jnp.full((_L,), _NEG, jnp.float32)

            def main_iter(i, ms):
                window = buf.at[pl.ds(i * unroll, slice_len)]
                for q in range(unroll // 4):
                    vs = tuple(
                        plsc.load_gather(window, [idx_main[4 * q + t]])
                        for t in range(4))
                    ms = _absorb4(ms, vs)
                return ms

            ms = lax.fori_loop(
                0, n_iters, main_iter, (negv, negv, negv, negv))

            for q in range(len(idx_tail) // 4):
                vs = tuple(
                    plsc.load_gather(buf, [idx_tail[4 * q + t]])
                    for t in range(4))
                ms = _absorb4(ms, vs)

            res = (ms[0] * wr[0] + ms[1] * wr[1]
                   + ms[2] * wr[2] + ms[3] * wr[3])
            outv[pl.ds(gi * chunk_rows, chunk_rows)] = res

            @pl.when(gi + 2 < n_groups)
            def _():
                start(gi + 2, buf, sem)

        def pair_body(p, carry):
            process(2 * p, buf0, sem0)
            process(2 * p + 1, buf1, sem1)
            return carry

        lax.fori_loop(0, n_groups // 2, pair_body, jnp.int32(0))

        pltpu.sync_copy(outv, out_hbm.at[pl.ds(wid * rows_per_w, rows_per_w)])

    return pl.kernel(
        body,
        out_type=jax.ShapeDtypeStruct((n_rows,), jnp.float32),
        mesh=mesh,
        compiler_params=pltpu.CompilerParams(needs_layout_passes=False),
        scratch_types=[
            pltpu.VMEM((_K, _L), jnp.float32),
            pltpu.VMEM((group_elems,), jnp.float32),
            pltpu.VMEM((group_elems,), jnp.float32),
            pltpu.VMEM((rows_per_w,), jnp.float32),
            pltpu.SemaphoreType.DMA,
            pltpu.SemaphoreType.DMA,
        ],
    )


def kernel(x, weights):
    b, c, h, w = x.shape
    n_rows = b * c
    row_len = h * w
    assert row_len % _L == 0 and n_rows % (_NW * _L) == 0
    xf = x.reshape(n_rows * row_len)
    wmat = jnp.broadcast_to(
        weights.reshape(_K, 1).astype(jnp.float32) / _K, (_K, _L))
    pool = _make_pool(n_rows, row_len, _L)
    out = pool(xf, wmat)
    return out.reshape(b, c, 1, 1)


# speedup vs baseline: 45.1817x; 5.4758x over previous
import jax
import jax.numpy as jnp
from jax import lax
from jax.experimental import pallas as pl
from jax.experimental.pallas import tpu as pltpu
from jax.experimental.pallas import tpu_sc as plsc

_K = 4
_L = 16
_NC = 2
_NS = 16
_NW = _NC * _NS
_NEG = -1.0e30


def _sort4(a, b, c, d):
    a, b = jnp.maximum(a, b), jnp.minimum(a, b)
    c, d = jnp.maximum(c, d), jnp.minimum(c, d)
    a, c = jnp.maximum(a, c), jnp.minimum(a, c)
    b, d = jnp.maximum(b, d), jnp.minimum(b, d)
    b, c = jnp.maximum(b, c), jnp.minimum(b, c)
    return a, b, c, d


def _merge44(ms, bs):
    a0, a1, a2, a3 = ms
    b0, b1, b2, b3 = bs
    m00 = jnp.minimum(a0, b0)
    m01 = jnp.minimum(a0, b1)
    m10 = jnp.minimum(a1, b0)
    m02 = jnp.minimum(a0, b2)
    m11 = jnp.minimum(a1, b1)
    m20 = jnp.minimum(a2, b0)
    c0 = jnp.maximum(a0, b0)
    c1 = jnp.maximum(m00, jnp.maximum(a1, b1))
    c2 = jnp.maximum(jnp.maximum(b2, a2), jnp.maximum(m01, m10))
    c3 = jnp.maximum(jnp.maximum(b3, a3),
                     jnp.maximum(m02, jnp.maximum(m11, m20)))
    return c0, c1, c2, c3


def _absorb4(ms, vs):
    return _merge44(ms, _sort4(*vs))


def _make_pool(n_b, n_c, n_hw):
    b_per_w = n_b // _NW
    ct_per_b = n_c // 128
    n_units = b_per_w * ct_per_b
    n_chunk = 4
    chunk_rows = n_hw // n_chunk
    n_tiles = chunk_rows // 8
    out_per_w = b_per_w * n_c
    mesh = plsc.VectorSubcoreMesh(core_axis_name="c", subcore_axis_name="s")

    def body(y_hbm, w_hbm, out_hbm, wv, buf0, buf1, outv, sem0, sem1):
        cid = lax.axis_index("c")
        sid = lax.axis_index("s")
        wid = sid * _NC + cid
        row_base = wid * b_per_w * n_hw

        pltpu.sync_copy(w_hbm, wv)
        wr = [wv[t, :] for t in range(_K)]
        negv = jnp.full((_L,), _NEG, jnp.float32)
        bufs = (buf0, buf1)
        sems = (sem0, sem1)

        def src(row0, c0, q):
            r = pl.multiple_of(row0 + q * chunk_rows, chunk_rows)
            c = pl.multiple_of(c0, 128)
            return y_hbm.at[pl.ds(r, chunk_rows), pl.ds(c, 128)]

        def advance(row0, c0):
            wrap = c0 + 128 >= n_c
            row0n = jnp.where(wrap, row0 + n_hw, row0)
            c0n = jnp.where(wrap, jnp.int32(0), c0 + 128)
            return row0n, c0n

        r00 = row_base + jnp.int32(0)
        c00 = jnp.int32(0)
        pltpu.async_copy(src(r00, c00, 0), buf0, sem0)
        pltpu.async_copy(src(r00, c00, 1), buf1, sem1)

        def unit_body(u, carry):
            row0, c0 = carry
            row0n, c0n = advance(row0, c0)
            ms = [(negv, negv, negv, negv) for _ in range(8)]
            for q in range(n_chunk):
                buf = bufs[q % 2]
                sem = sems[q % 2]
                pltpu.make_async_copy(src(r00, c00, 0), buf, sem).wait()
                for p in range(8):
                    def pass_body(t, m, _p=p, _buf=buf):
                        vs = []
                        for w8 in range(8):
                            vs.append(_buf[t * 8 + w8,
                                           pl.ds(_p * _L, _L)])
                            if len(vs) == 4:
                                m = _absorb4(m, tuple(vs))
                                vs = []
                        return m
                    ms[p] = lax.fori_loop(0, n_tiles, pass_body, ms[p])
                nq = q + 2
                if nq < n_chunk:
                    pltpu.async_copy(src(row0, c0, nq), buf, sem)
                else:
                    @pl.when(u + 1 < n_units)
                    def _():
                        pltpu.async_copy(src(row0n, c0n, nq - n_chunk),
                                         buf, sem)
            obase = u * 128
            for p in range(8):
                m0, m1, m2, m3 = ms[p]
                res = m0 * wr[0] + m1 * wr[1] + m2 * wr[2] + m3 * wr[3]
                outv[pl.ds(obase + p * _L, _L)] = res
            return row0n, c0n

        lax.fori_loop(0, n_units, unit_body, (r00, c00))

        pltpu.sync_copy(outv, out_hbm.at[pl.ds(wid * out_per_w, out_per_w)])

    return pl.kernel(
        body,
        out_type=jax.ShapeDtypeStruct((n_b * n_c,), jnp.float32),
        mesh=mesh,
        compiler_params=pltpu.CompilerParams(needs_layout_passes=False),
        scratch_types=[
            pltpu.VMEM((_K, _L), jnp.float32),
            pltpu.VMEM((chunk_rows, 128), jnp.float32),
            pltpu.VMEM((chunk_rows, 128), jnp.float32),
            pltpu.VMEM((out_per_w,), jnp.float32),
            pltpu.SemaphoreType.DMA,
            pltpu.SemaphoreType.DMA,
        ],
    )


def kernel(x, weights):
    b, c, h, w = x.shape
    n_hw = h * w
    assert c % 128 == 0 and b % _NW == 0 and n_hw % 32 == 0
    y = x.transpose(0, 2, 3, 1).reshape(b * n_hw, c)
    wmat = jnp.broadcast_to(
        weights.reshape(_K, 1).astype(jnp.float32) / _K, (_K, _L))
    pool = _make_pool(b, c, n_hw)
    out = pool(y, wmat)
    return out.reshape(b, c, 1, 1)


# speedup vs baseline: 47.3435x; 1.0478x over previous
import jax
import jax.numpy as jnp
from jax import lax
from jax.experimental import pallas as pl
from jax.experimental.pallas import tpu as pltpu
from jax.experimental.pallas import tpu_sc as plsc

_K = 4
_L = 16
_NC = 2
_NS = 16
_NW = _NC * _NS
_NEG = -1.0e30


def _sort4(a, b, c, d):
    a, b = jnp.maximum(a, b), jnp.minimum(a, b)
    c, d = jnp.maximum(c, d), jnp.minimum(c, d)
    a, c = jnp.maximum(a, c), jnp.minimum(a, c)
    b, d = jnp.maximum(b, d), jnp.minimum(b, d)
    b, c = jnp.maximum(b, c), jnp.minimum(b, c)
    return a, b, c, d


def _merge44(ms, bs):
    a0, a1, a2, a3 = ms
    b0, b1, b2, b3 = bs
    m00 = jnp.minimum(a0, b0)
    m01 = jnp.minimum(a0, b1)
    m10 = jnp.minimum(a1, b0)
    m02 = jnp.minimum(a0, b2)
    m11 = jnp.minimum(a1, b1)
    m20 = jnp.minimum(a2, b0)
    c0 = jnp.maximum(a0, b0)
    c1 = jnp.maximum(m00, jnp.maximum(a1, b1))
    c2 = jnp.maximum(jnp.maximum(b2, a2), jnp.maximum(m01, m10))
    c3 = jnp.maximum(jnp.maximum(b3, a3),
                     jnp.maximum(m02, jnp.maximum(m11, m20)))
    return c0, c1, c2, c3


def _absorb4(ms, vs):
    return _merge44(ms, _sort4(*vs))


def _make_pool(n_b, n_c, n_c_sc, n_hw):
    b_per_w = n_b // _NW
    ct_per_b = n_c_sc // 128
    n_units = b_per_w * ct_per_b
    n_chunk = 4
    chunk_rows = n_hw // n_chunk
    n_tiles = chunk_rows // 8
    out_per_w = b_per_w * n_c_sc
    mesh = plsc.VectorSubcoreMesh(core_axis_name="c", subcore_axis_name="s")

    def body(y_hbm, w_hbm, out_hbm, wv, buf0, buf1, outv, sem0, sem1):
        cid = lax.axis_index("c")
        sid = lax.axis_index("s")
        wid = sid * _NC + cid
        row_base = wid * b_per_w * n_hw

        pltpu.sync_copy(w_hbm, wv)
        wr = [wv[t, :] for t in range(_K)]
        negv = jnp.full((_L,), _NEG, jnp.float32)
        bufs = (buf0, buf1)
        sems = (sem0, sem1)

        def src(row0, c0, q):
            r = pl.multiple_of(row0 + q * chunk_rows, chunk_rows)
            c = pl.multiple_of(c0, 128)
            return y_hbm.at[pl.ds(r, chunk_rows), pl.ds(c, 128)]

        def advance(row0, c0):
            wrap = c0 + 128 >= n_c_sc
            row0n = jnp.where(wrap, row0 + n_hw, row0)
            c0n = jnp.where(wrap, jnp.int32(0), c0 + 128)
            return row0n, c0n

        r00 = row_base + jnp.int32(0)
        c00 = jnp.int32(0)
        pltpu.async_copy(src(r00, c00, 0), buf0, sem0)
        pltpu.async_copy(src(r00, c00, 1), buf1, sem1)

        def unit_body(u, carry):
            row0, c0 = carry
            row0n, c0n = advance(row0, c0)
            ms = [(negv, negv, negv, negv) for _ in range(8)]
            for q in range(n_chunk):
                buf = bufs[q % 2]
                sem = sems[q % 2]
                pltpu.make_async_copy(src(r00, c00, 0), buf, sem).wait()
                for p in range(8):
                    def pass_body(t, m, _p=p, _buf=buf):
                        vs = []
                        for w8 in range(8):
                            vs.append(_buf[t * 8 + w8,
                                           pl.ds(_p * _L, _L)])
                            if len(vs) == 4:
                                m = _absorb4(m, tuple(vs))
                                vs = []
                        return m
                    ms[p] = lax.fori_loop(0, n_tiles, pass_body, ms[p])
                nq = q + 2
                if nq < n_chunk:
                    pltpu.async_copy(src(row0, c0, nq), buf, sem)
                else:
                    @pl.when(u + 1 < n_units)
                    def _():
                        pltpu.async_copy(src(row0n, c0n, nq - n_chunk),
                                         buf, sem)
            obase = u * 128
            for p in range(8):
                m0, m1, m2, m3 = ms[p]
                res = m0 * wr[0] + m1 * wr[1] + m2 * wr[2] + m3 * wr[3]
                outv[pl.ds(obase + p * _L, _L)] = res
            return row0n, c0n

        lax.fori_loop(0, n_units, unit_body, (r00, c00))

        pltpu.sync_copy(outv, out_hbm.at[pl.ds(wid * out_per_w, out_per_w)])

    return pl.kernel(
        body,
        out_type=jax.ShapeDtypeStruct((n_b * n_c_sc,), jnp.float32),
        mesh=mesh,
        compiler_params=pltpu.CompilerParams(needs_layout_passes=False),
        scratch_types=[
            pltpu.VMEM((_K, _L), jnp.float32),
            pltpu.VMEM((chunk_rows, 128), jnp.float32),
            pltpu.VMEM((chunk_rows, 128), jnp.float32),
            pltpu.VMEM((out_per_w,), jnp.float32),
            pltpu.SemaphoreType.DMA,
            pltpu.SemaphoreType.DMA,
        ],
    )


def _tc_block(w_ref, y_ref, o_ref):
    x = y_ref[...]
    negv = jnp.full(x.shape, _NEG, x.dtype)
    p = jnp.zeros((1, x.shape[1]), jnp.int32)
    vals = [jnp.zeros((1, x.shape[1]), jnp.float32) for _ in range(_K)]
    m = x
    for _ in range(_K):
        g = jnp.max(m, axis=0, keepdims=True)
        eq = m == g
        cnt = jnp.sum(eq.astype(jnp.int32), axis=0, keepdims=True)
        pn = p + cnt
        for r in range(_K):
            vals[r] = jnp.where((r >= p) & (r < pn), g, vals[r])
        m = jnp.where(eq, negv, m)
        p = pn
    acc = vals[0] * w_ref[0]
    for r in range(1, _K):
        acc = acc + vals[r] * w_ref[r]
    o_ref[...] = acc.reshape(o_ref.shape)


def _make_tc_pool(n_b, n_c_tc, n_hw, c_off):
    grid = (n_b, n_c_tc // 128)
    return pl.pallas_call(
        _tc_block,
        grid=grid,
        in_specs=[
            pl.BlockSpec(memory_space=pltpu.SMEM),
            pl.BlockSpec((n_hw, 128),
                         lambda i, j: (i, c_off // 128 + j)),
        ],
        out_specs=pl.BlockSpec((1, 1, 128), lambda i, j: (i, 0, j)),
        out_shape=jax.ShapeDtypeStruct((n_b, 1, n_c_tc), jnp.float32),
        compiler_params=pltpu.CompilerParams(
            dimension_semantics=("arbitrary", "arbitrary")),
    )


_C_SC = 512


def kernel(x, weights):
    b, c, h, w = x.shape
    n_hw = h * w
    assert c % 128 == 0 and b % _NW == 0 and n_hw % 32 == 0
    y = x.transpose(0, 2, 3, 1).reshape(b * n_hw, c)
    wmat = jnp.broadcast_to(
        weights.reshape(_K, 1).astype(jnp.float32) / _K, (_K, _L))
    n_c_sc = _C_SC if 0 < _C_SC < c else c
    pool = _make_pool(b, c, n_c_sc, n_hw)
    out_sc = pool(y, wmat).reshape(b, n_c_sc)
    if n_c_sc < c:
        wvec = weights.reshape(_K).astype(jnp.float32) / _K
        tc_pool = _make_tc_pool(b, c - n_c_sc, n_hw, n_c_sc)
        out_tc = tc_pool(wvec, y).reshape(b, c - n_c_sc)
        out = jnp.concatenate([out_sc, out_tc], axis=1)
    else:
        out = out_sc
    return out.reshape(b, c, 1, 1)


# speedup vs baseline: 55.9670x; 1.1821x over previous
import jax
import jax.numpy as jnp
from jax import lax
from jax.experimental import pallas as pl
from jax.experimental.pallas import tpu as pltpu
from jax.experimental.pallas import tpu_sc as plsc

_K = 4
_L = 16
_NC = 2
_NS = 16
_NW = _NC * _NS
_NEG = -1.0e30


def _sort4(a, b, c, d):
    a, b = jnp.maximum(a, b), jnp.minimum(a, b)
    c, d = jnp.maximum(c, d), jnp.minimum(c, d)
    a, c = jnp.maximum(a, c), jnp.minimum(a, c)
    b, d = jnp.maximum(b, d), jnp.minimum(b, d)
    b, c = jnp.maximum(b, c), jnp.minimum(b, c)
    return a, b, c, d


def _merge44(ms, bs):
    a0, a1, a2, a3 = ms
    b0, b1, b2, b3 = bs
    m00 = jnp.minimum(a0, b0)
    m01 = jnp.minimum(a0, b1)
    m10 = jnp.minimum(a1, b0)
    m02 = jnp.minimum(a0, b2)
    m11 = jnp.minimum(a1, b1)
    m20 = jnp.minimum(a2, b0)
    c0 = jnp.maximum(a0, b0)
    c1 = jnp.maximum(m00, jnp.maximum(a1, b1))
    c2 = jnp.maximum(jnp.maximum(b2, a2), jnp.maximum(m01, m10))
    c3 = jnp.maximum(jnp.maximum(b3, a3),
                     jnp.maximum(m02, jnp.maximum(m11, m20)))
    return c0, c1, c2, c3


def _absorb4(ms, vs):
    return _merge44(ms, _sort4(*vs))


def _make_pool(n_b, n_c, n_c_sc, n_hw):
    b_per_w = n_b // _NW
    ct_per_b = n_c_sc // 128
    n_units = b_per_w * ct_per_b
    n_chunk = 4
    chunk_rows = n_hw // n_chunk
    n_tiles = chunk_rows // 8
    out_per_w = b_per_w * n_c_sc
    mesh = plsc.VectorSubcoreMesh(core_axis_name="c", subcore_axis_name="s")

    def body(y_hbm, w_hbm, out_hbm, wv, buf0, buf1, outv, sem0, sem1):
        cid = lax.axis_index("c")
        sid = lax.axis_index("s")
        wid = sid * _NC + cid
        row_base = wid * b_per_w * n_hw

        pltpu.sync_copy(w_hbm, wv)
        wr = [wv[t, :] for t in range(_K)]
        negv = jnp.full((_L,), _NEG, jnp.float32)
        bufs = (buf0, buf1)
        sems = (sem0, sem1)

        def src(row0, c0, q):
            r = pl.multiple_of(row0 + q * chunk_rows, chunk_rows)
            c = pl.multiple_of(c0, 128)
            return y_hbm.at[pl.ds(r, chunk_rows), pl.ds(c, 128)]

        def advance(row0, c0):
            wrap = c0 + 128 >= n_c_sc
            row0n = jnp.where(wrap, row0 + n_hw, row0)
            c0n = jnp.where(wrap, jnp.int32(0), c0 + 128)
            return row0n, c0n

        r00 = row_base + jnp.int32(0)
        c00 = jnp.int32(0)
        pltpu.async_copy(src(r00, c00, 0), buf0, sem0)
        pltpu.async_copy(src(r00, c00, 1), buf1, sem1)

        def unit_body(u, carry):
            row0, c0 = carry
            row0n, c0n = advance(row0, c0)
            ms = [(negv, negv, negv, negv) for _ in range(8)]
            for q in range(n_chunk):
                buf = bufs[q % 2]
                sem = sems[q % 2]
                pltpu.make_async_copy(src(r00, c00, 0), buf, sem).wait()
                for p in range(8):
                    def pass_body(t, m, _p=p, _buf=buf):
                        vs = []
                        for w8 in range(8):
                            vs.append(_buf[t * 8 + w8,
                                           pl.ds(_p * _L, _L)])
                            if len(vs) == 4:
                                m = _absorb4(m, tuple(vs))
                                vs = []
                        return m
                    ms[p] = lax.fori_loop(0, n_tiles, pass_body, ms[p])
                nq = q + 2
                if nq < n_chunk:
                    pltpu.async_copy(src(row0, c0, nq), buf, sem)
                else:
                    @pl.when(u + 1 < n_units)
                    def _():
                        pltpu.async_copy(src(row0n, c0n, nq - n_chunk),
                                         buf, sem)
            obase = u * 128
            for p in range(8):
                m0, m1, m2, m3 = ms[p]
                res = m0 * wr[0] + m1 * wr[1] + m2 * wr[2] + m3 * wr[3]
                outv[pl.ds(obase + p * _L, _L)] = res
            return row0n, c0n

        lax.fori_loop(0, n_units, unit_body, (r00, c00))

        pltpu.sync_copy(outv, out_hbm.at[pl.ds(wid * out_per_w, out_per_w)])

    return pl.kernel(
        body,
        out_type=jax.ShapeDtypeStruct((n_b * n_c_sc,), jnp.float32),
        mesh=mesh,
        compiler_params=pltpu.CompilerParams(needs_layout_passes=False),
        scratch_types=[
            pltpu.VMEM((_K, _L), jnp.float32),
            pltpu.VMEM((chunk_rows, 128), jnp.float32),
            pltpu.VMEM((chunk_rows, 128), jnp.float32),
            pltpu.VMEM((out_per_w,), jnp.float32),
            pltpu.SemaphoreType.DMA,
            pltpu.SemaphoreType.DMA,
        ],
    )


def _tc_block(w_ref, y_ref, o_ref):
    n_hw = y_ref.shape[0]
    negv = jnp.full((8, 128), _NEG, jnp.float32)

    def body(i, ms):
        vs = tuple(y_ref[pl.ds(i * 32 + t * 8, 8), :] for t in range(4))
        return _absorb4(ms, vs)

    ms = lax.fori_loop(0, n_hw // 32, body, (negv, negv, negv, negv))
    for h in (4, 2, 1):
        a = tuple(m[:h] for m in ms)
        b = tuple(m[h:2 * h] for m in ms)
        ms = _merge44(a, b)
    acc = ms[0] * w_ref[0]
    for r in range(1, _K):
        acc = acc + ms[r] * w_ref[r]
    o_ref[...] = acc.reshape(o_ref.shape)


def _make_tc_pool(n_b, n_c_tc, n_hw, c_off):
    grid = (n_b, n_c_tc // 128)
    return pl.pallas_call(
        _tc_block,
        grid=grid,
        in_specs=[
            pl.BlockSpec(memory_space=pltpu.SMEM),
            pl.BlockSpec((n_hw, 128),
                         lambda i, j: (i, c_off // 128 + j)),
        ],
        out_specs=pl.BlockSpec((1, 1, 128), lambda i, j: (i, 0, j)),
        out_shape=jax.ShapeDtypeStruct((n_b, 1, n_c_tc), jnp.float32),
        compiler_params=pltpu.CompilerParams(
            dimension_semantics=("arbitrary", "arbitrary")),
    )


_C_SC = 512


def kernel(x, weights):
    b, c, h, w = x.shape
    n_hw = h * w
    assert c % 128 == 0 and b % _NW == 0 and n_hw % 32 == 0
    y = x.transpose(0, 2, 3, 1).reshape(b * n_hw, c)
    wmat = jnp.broadcast_to(
        weights.reshape(_K, 1).astype(jnp.float32) / _K, (_K, _L))
    n_c_sc = _C_SC if 0 < _C_SC < c else c
    pool = _make_pool(b, c, n_c_sc, n_hw)
    out_sc = pool(y, wmat).reshape(b, n_c_sc)
    if n_c_sc < c:
        wvec = weights.reshape(_K).astype(jnp.float32) / _K
        tc_pool = _make_tc_pool(b, c - n_c_sc, n_hw, n_c_sc)
        out_tc = tc_pool(wvec, y).reshape(b, c - n_c_sc)
        out = jnp.concatenate([out_sc, out_tc], axis=1)
    else:
        out = out_sc
    return out.reshape(b, c, 1, 1)


# speedup vs baseline: 59.2944x; 1.0595x over previous
import jax
import jax.numpy as jnp
from jax import lax
from jax.experimental import pallas as pl
from jax.experimental.pallas import tpu as pltpu
from jax.experimental.pallas import tpu_sc as plsc

_K = 4
_L = 16
_NC = 2
_NS = 16
_NW = _NC * _NS
_NEG = -1.0e30


def _sort4(a, b, c, d):
    a, b = jnp.maximum(a, b), jnp.minimum(a, b)
    c, d = jnp.maximum(c, d), jnp.minimum(c, d)
    a, c = jnp.maximum(a, c), jnp.minimum(a, c)
    b, d = jnp.maximum(b, d), jnp.minimum(b, d)
    b, c = jnp.maximum(b, c), jnp.minimum(b, c)
    return a, b, c, d


def _merge44(ms, bs):
    a0, a1, a2, a3 = ms
    b0, b1, b2, b3 = bs
    m00 = jnp.minimum(a0, b0)
    m01 = jnp.minimum(a0, b1)
    m10 = jnp.minimum(a1, b0)
    m02 = jnp.minimum(a0, b2)
    m11 = jnp.minimum(a1, b1)
    m20 = jnp.minimum(a2, b0)
    c0 = jnp.maximum(a0, b0)
    c1 = jnp.maximum(m00, jnp.maximum(a1, b1))
    c2 = jnp.maximum(jnp.maximum(b2, a2), jnp.maximum(m01, m10))
    c3 = jnp.maximum(jnp.maximum(b3, a3),
                     jnp.maximum(m02, jnp.maximum(m11, m20)))
    return c0, c1, c2, c3


def _absorb4(ms, vs):
    return _merge44(ms, _sort4(*vs))


def _make_pool(n_b, n_c, n_c_sc, n_hw):
    b_per_w = n_b // _NW
    ct_per_b = n_c_sc // 128
    n_units = b_per_w * ct_per_b
    n_chunk = 4
    chunk_rows = n_hw // n_chunk
    n_tiles = chunk_rows // 8
    out_per_w = b_per_w * n_c_sc
    mesh = plsc.VectorSubcoreMesh(core_axis_name="c", subcore_axis_name="s")

    def body(y_hbm, w_hbm, out_hbm, wv, buf0, buf1, outv, sem0, sem1):
        cid = lax.axis_index("c")
        sid = lax.axis_index("s")
        wid = sid * _NC + cid
        row_base = wid * b_per_w * n_hw

        pltpu.sync_copy(w_hbm, wv)
        wr = [wv[t, :] for t in range(_K)]
        negv = jnp.full((_L,), _NEG, jnp.float32)
        bufs = (buf0, buf1)
        sems = (sem0, sem1)

        def src(row0, c0, q):
            r = pl.multiple_of(row0 + q * chunk_rows, chunk_rows)
            c = pl.multiple_of(c0, 128)
            return y_hbm.at[pl.ds(r, chunk_rows), pl.ds(c, 128)]

        def advance(row0, c0):
            wrap = c0 + 128 >= n_c_sc
            row0n = jnp.where(wrap, row0 + n_hw, row0)
            c0n = jnp.where(wrap, jnp.int32(0), c0 + 128)
            return row0n, c0n

        r00 = row_base + jnp.int32(0)
        c00 = jnp.int32(0)
        pltpu.async_copy(src(r00, c00, 0), buf0, sem0)
        pltpu.async_copy(src(r00, c00, 1), buf1, sem1)

        def unit_body(u, carry):
            row0, c0 = carry
            row0n, c0n = advance(row0, c0)
            ms = [(negv, negv, negv, negv) for _ in range(8)]
            for q in range(n_chunk):
                buf = bufs[q % 2]
                sem = sems[q % 2]
                pltpu.make_async_copy(src(r00, c00, 0), buf, sem).wait()
                for p in range(8):
                    def pass_body(t, m, _p=p, _buf=buf):
                        vs = []
                        for w8 in range(8):
                            vs.append(_buf[t * 8 + w8,
                                           pl.ds(_p * _L, _L)])
                            if len(vs) == 4:
                                m = _absorb4(m, tuple(vs))
                                vs = []
                        return m
                    ms[p] = lax.fori_loop(0, n_tiles, pass_body, ms[p])
                nq = q + 2
                if nq < n_chunk:
                    pltpu.async_copy(src(row0, c0, nq), buf, sem)
                else:
                    @pl.when(u + 1 < n_units)
                    def _():
                        pltpu.async_copy(src(row0n, c0n, nq - n_chunk),
                                         buf, sem)
            obase = u * 128
            for p in range(8):
                m0, m1, m2, m3 = ms[p]
                res = m0 * wr[0] + m1 * wr[1] + m2 * wr[2] + m3 * wr[3]
                outv[pl.ds(obase + p * _L, _L)] = res
            return row0n, c0n

        lax.fori_loop(0, n_units, unit_body, (r00, c00))

        pltpu.sync_copy(outv, out_hbm.at[pl.ds(wid * out_per_w, out_per_w)])

    return pl.kernel(
        body,
        out_type=jax.ShapeDtypeStruct((n_b * n_c_sc,), jnp.float32),
        mesh=mesh,
        compiler_params=pltpu.CompilerParams(needs_layout_passes=False),
        scratch_types=[
            pltpu.VMEM((_K, _L), jnp.float32),
            pltpu.VMEM((chunk_rows, 128), jnp.float32),
            pltpu.VMEM((chunk_rows, 128), jnp.float32),
            pltpu.VMEM((out_per_w,), jnp.float32),
            pltpu.SemaphoreType.DMA,
            pltpu.SemaphoreType.DMA,
        ],
    )


def _tc_block(w_ref, y_ref, o_ref):
    n_hw = y_ref.shape[0]
    negv = jnp.full((8, 128), _NEG, jnp.float32)

    def body(i, st):
        a, b = st
        vsa = tuple(y_ref[pl.ds(i * 64 + t * 8, 8), :] for t in range(4))
        vsb = tuple(y_ref[pl.ds(i * 64 + 32 + t * 8, 8), :]
                    for t in range(4))
        return (_absorb4(a, vsa), _absorb4(b, vsb))

    neg4 = (negv, negv, negv, negv)
    msa, msb = lax.fori_loop(0, n_hw // 64, body, (neg4, neg4))
    ms = _merge44(msa, msb)
    for h in (4, 2, 1):
        a = tuple(m[:h] for m in ms)
        b = tuple(m[h:2 * h] for m in ms)
        ms = _merge44(a, b)
    acc = ms[0] * w_ref[0]
    for r in range(1, _K):
        acc = acc + ms[r] * w_ref[r]
    o_ref[...] = acc.reshape(o_ref.shape)


def _make_tc_pool(n_b, n_c_tc, n_hw, c_off):
    grid = (n_b, n_c_tc // 128)
    return pl.pallas_call(
        _tc_block,
        grid=grid,
        in_specs=[
            pl.BlockSpec(memory_space=pltpu.SMEM),
            pl.BlockSpec((n_hw, 128),
                         lambda i, j: (i, c_off // 128 + j)),
        ],
        out_specs=pl.BlockSpec((1, 1, 128), lambda i, j: (i, 0, j)),
        out_shape=jax.ShapeDtypeStruct((n_b, 1, n_c_tc), jnp.float32),
        compiler_params=pltpu.CompilerParams(
            dimension_semantics=("arbitrary", "arbitrary")),
    )


_C_SC = 512


def kernel(x, weights):
    b, c, h, w = x.shape
    n_hw = h * w
    assert c % 128 == 0 and b % _NW == 0 and n_hw % 32 == 0
    y = x.transpose(0, 2, 3, 1).reshape(b * n_hw, c)
    wmat = jnp.broadcast_to(
        weights.reshape(_K, 1).astype(jnp.float32) / _K, (_K, _L))
    n_c_sc = _C_SC if 0 < _C_SC < c else c
    pool = _make_pool(b, c, n_c_sc, n_hw)
    out_sc = pool(y, wmat).reshape(b, n_c_sc)
    if n_c_sc < c:
        wvec = weights.reshape(_K).astype(jnp.float32) / _K
        tc_pool = _make_tc_pool(b, c - n_c_sc, n_hw, n_c_sc)
        out_tc = tc_pool(wvec, y).reshape(b, c - n_c_sc)
        out = jnp.concatenate([out_sc, out_tc], axis=1)
    else:
        out = out_sc
    return out.reshape(b, c, 1, 1)


# speedup vs baseline: 60.0017x; 1.0119x over previous
import jax
import jax.numpy as jnp
from jax import lax
from jax.experimental import pallas as pl
from jax.experimental.pallas import tpu as pltpu
from jax.experimental.pallas import tpu_sc as plsc

_K = 4
_L = 16
_NC = 2
_NS = 16
_NW = _NC * _NS
_NEG = -1.0e30


def _sort4(a, b, c, d):
    a, b = jnp.maximum(a, b), jnp.minimum(a, b)
    c, d = jnp.maximum(c, d), jnp.minimum(c, d)
    a, c = jnp.maximum(a, c), jnp.minimum(a, c)
    b, d = jnp.maximum(b, d), jnp.minimum(b, d)
    b, c = jnp.maximum(b, c), jnp.minimum(b, c)
    return a, b, c, d


def _merge44(ms, bs):
    a0, a1, a2, a3 = ms
    b0, b1, b2, b3 = bs
    m00 = jnp.minimum(a0, b0)
    m01 = jnp.minimum(a0, b1)
    m10 = jnp.minimum(a1, b0)
    m02 = jnp.minimum(a0, b2)
    m11 = jnp.minimum(a1, b1)
    m20 = jnp.minimum(a2, b0)
    c0 = jnp.maximum(a0, b0)
    c1 = jnp.maximum(m00, jnp.maximum(a1, b1))
    c2 = jnp.maximum(jnp.maximum(b2, a2), jnp.maximum(m01, m10))
    c3 = jnp.maximum(jnp.maximum(b3, a3),
                     jnp.maximum(m02, jnp.maximum(m11, m20)))
    return c0, c1, c2, c3


def _absorb4(ms, vs):
    return _merge44(ms, _sort4(*vs))


def _make_pool(n_b, n_c, n_c_sc, n_hw):
    b_per_w = n_b // _NW
    ct_per_b = n_c_sc // 128
    n_units = b_per_w * ct_per_b
    n_chunk = 4
    chunk_rows = n_hw // n_chunk
    n_tiles = chunk_rows // 8
    out_per_w = b_per_w * n_c_sc
    mesh = plsc.VectorSubcoreMesh(core_axis_name="c", subcore_axis_name="s")

    def body(y_hbm, w_hbm, out_hbm, wv, buf0, buf1, outv, sem0, sem1):
        cid = lax.axis_index("c")
        sid = lax.axis_index("s")
        wid = sid * _NC + cid
        row_base = wid * b_per_w * n_hw

        pltpu.sync_copy(w_hbm, wv)
        wr = [wv[t, :] for t in range(_K)]
        negv = jnp.full((_L,), _NEG, jnp.float32)
        bufs = (buf0, buf1)
        sems = (sem0, sem1)

        def src(row0, c0, q):
            r = pl.multiple_of(row0 + q * chunk_rows, chunk_rows)
            c = pl.multiple_of(c0, 128)
            return y_hbm.at[pl.ds(r, chunk_rows), pl.ds(c, 128)]

        def advance(row0, c0):
            wrap = c0 + 128 >= n_c_sc
            row0n = jnp.where(wrap, row0 + n_hw, row0)
            c0n = jnp.where(wrap, jnp.int32(0), c0 + 128)
            return row0n, c0n

        r00 = row_base + jnp.int32(0)
        c00 = jnp.int32(0)
        pltpu.async_copy(src(r00, c00, 0), buf0, sem0)
        pltpu.async_copy(src(r00, c00, 1), buf1, sem1)

        def unit_body(u, carry):
            row0, c0 = carry
            row0n, c0n = advance(row0, c0)
            ms = [(negv, negv, negv, negv) for _ in range(8)]
            for q in range(n_chunk):
                buf = bufs[q % 2]
                sem = sems[q % 2]
                pltpu.make_async_copy(src(r00, c00, 0), buf, sem).wait()
                for p in range(8):
                    def pass_body(t, m, _p=p, _buf=buf):
                        vs = []
                        for w8 in range(8):
                            vs.append(_buf[t * 8 + w8,
                                           pl.ds(_p * _L, _L)])
                            if len(vs) == 4:
                                m = _absorb4(m, tuple(vs))
                                vs = []
                        return m
                    ms[p] = lax.fori_loop(0, n_tiles, pass_body, ms[p])
                nq = q + 2
                if nq < n_chunk:
                    pltpu.async_copy(src(row0, c0, nq), buf, sem)
                else:
                    @pl.when(u + 1 < n_units)
                    def _():
                        pltpu.async_copy(src(row0n, c0n, nq - n_chunk),
                                         buf, sem)
            obase = u * 128
            for p in range(8):
                m0, m1, m2, m3 = ms[p]
                res = m0 * wr[0] + m1 * wr[1] + m2 * wr[2] + m3 * wr[3]
                outv[pl.ds(obase + p * _L, _L)] = res
            return row0n, c0n

        lax.fori_loop(0, n_units, unit_body, (r00, c00))

        pltpu.sync_copy(outv, out_hbm.at[pl.ds(wid * out_per_w, out_per_w)])

    return pl.kernel(
        body,
        out_type=jax.ShapeDtypeStruct((n_b * n_c_sc,), jnp.float32),
        mesh=mesh,
        compiler_params=pltpu.CompilerParams(needs_layout_passes=False),
        scratch_types=[
            pltpu.VMEM((_K, _L), jnp.float32),
            pltpu.VMEM((chunk_rows, 128), jnp.float32),
            pltpu.VMEM((chunk_rows, 128), jnp.float32),
            pltpu.VMEM((out_per_w,), jnp.float32),
            pltpu.SemaphoreType.DMA,
            pltpu.SemaphoreType.DMA,
        ],
    )


def _tc_block(w_ref, y_ref, o_ref):
    n_hw = y_ref.shape[0]
    negv = jnp.full((8, 128), _NEG, jnp.float32)

    n_acc = 4
    rows_per_iter = 32 * n_acc

    def body(i, st):
        out = []
        for a in range(n_acc):
            base = i * rows_per_iter + a * 32
            vs = tuple(y_ref[pl.ds(base + t * 8, 8), :] for t in range(4))
            out.append(_absorb4(st[a], vs))
        return tuple(out)

    neg4 = (negv, negv, negv, negv)
    sts = lax.fori_loop(0, n_hw // rows_per_iter, body, (neg4,) * n_acc)
    while len(sts) > 1:
        sts = tuple(_merge44(sts[2 * i], sts[2 * i + 1])
                    for i in range(len(sts) // 2))
    ms = sts[0]
    for h in (4, 2, 1):
        a = tuple(m[:h] for m in ms)
        b = tuple(m[h:2 * h] for m in ms)
        ms = _merge44(a, b)
    acc = ms[0] * w_ref[0]
    for r in range(1, _K):
        acc = acc + ms[r] * w_ref[r]
    o_ref[...] = acc.reshape(o_ref.shape)


def _make_tc_pool(n_b, n_c_tc, n_hw, c_off):
    grid = (n_b, n_c_tc // 128)
    return pl.pallas_call(
        _tc_block,
        grid=grid,
        in_specs=[
            pl.BlockSpec(memory_space=pltpu.SMEM),
            pl.BlockSpec((n_hw, 128),
                         lambda i, j: (i, c_off // 128 + j)),
        ],
        out_specs=pl.BlockSpec((1, 1, 128), lambda i, j: (i, 0, j)),
        out_shape=jax.ShapeDtypeStruct((n_b, 1, n_c_tc), jnp.float32),
        compiler_params=pltpu.CompilerParams(
            dimension_semantics=("arbitrary", "arbitrary")),
    )


_C_SC = 512


def kernel(x, weights):
    b, c, h, w = x.shape
    n_hw = h * w
    assert c % 128 == 0 and b % _NW == 0 and n_hw % 32 == 0
    y = x.transpose(0, 2, 3, 1).reshape(b * n_hw, c)
    wmat = jnp.broadcast_to(
        weights.reshape(_K, 1).astype(jnp.float32) / _K, (_K, _L))
    n_c_sc = _C_SC if 0 < _C_SC < c else c
    pool = _make_pool(b, c, n_c_sc, n_hw)
    out_sc = pool(y, wmat).reshape(b, n_c_sc)
    if n_c_sc < c:
        wvec = weights.reshape(_K).astype(jnp.float32) / _K
        tc_pool = _make_tc_pool(b, c - n_c_sc, n_hw, n_c_sc)
        out_tc = tc_pool(wvec, y).reshape(b, c - n_c_sc)
        out = jnp.concatenate([out_sc, out_tc], axis=1)
    else:
        out = out_sc
    return out.reshape(b, c, 1, 1)


# speedup vs baseline: 72.3907x; 1.2065x over previous
import jax
import jax.numpy as jnp
from jax import lax
from jax.experimental import pallas as pl
from jax.experimental.pallas import tpu as pltpu
from jax.experimental.pallas import tpu_sc as plsc

_K = 4
_L = 16
_NC = 2
_NS = 16
_NW = _NC * _NS
_NEG = -1.0e30


def _sort4(a, b, c, d):
    a, b = jnp.maximum(a, b), jnp.minimum(a, b)
    c, d = jnp.maximum(c, d), jnp.minimum(c, d)
    a, c = jnp.maximum(a, c), jnp.minimum(a, c)
    b, d = jnp.maximum(b, d), jnp.minimum(b, d)
    b, c = jnp.maximum(b, c), jnp.minimum(b, c)
    return a, b, c, d


def _merge44(ms, bs):
    a0, a1, a2, a3 = ms
    b0, b1, b2, b3 = bs
    m00 = jnp.minimum(a0, b0)
    m01 = jnp.minimum(a0, b1)
    m10 = jnp.minimum(a1, b0)
    m02 = jnp.minimum(a0, b2)
    m11 = jnp.minimum(a1, b1)
    m20 = jnp.minimum(a2, b0)
    c0 = jnp.maximum(a0, b0)
    c1 = jnp.maximum(m00, jnp.maximum(a1, b1))
    c2 = jnp.maximum(jnp.maximum(b2, a2), jnp.maximum(m01, m10))
    c3 = jnp.maximum(jnp.maximum(b3, a3),
                     jnp.maximum(m02, jnp.maximum(m11, m20)))
    return c0, c1, c2, c3


def _absorb4(ms, vs):
    return _merge44(ms, _sort4(*vs))


def _make_pool(n_b, n_c, n_c_sc, n_hw):
    b_per_w = n_b // _NW
    ct_per_b = n_c_sc // 128
    n_units = b_per_w * ct_per_b
    n_chunk = 4
    chunk_rows = n_hw // n_chunk
    n_tiles = chunk_rows // 8
    out_per_w = b_per_w * n_c_sc
    mesh = plsc.VectorSubcoreMesh(core_axis_name="c", subcore_axis_name="s")

    def body(y_hbm, w_hbm, out_hbm, wv, buf0, buf1, outv, sem0, sem1):
        cid = lax.axis_index("c")
        sid = lax.axis_index("s")
        wid = sid * _NC + cid
        row_base = wid * b_per_w * n_hw

        pltpu.sync_copy(w_hbm, wv)
        wr = [wv[t, :] for t in range(_K)]
        negv = jnp.full((_L,), _NEG, jnp.float32)
        bufs = (buf0, buf1)
        sems = (sem0, sem1)

        def src(row0, c0, q):
            r = pl.multiple_of(row0 + q * chunk_rows, chunk_rows)
            c = pl.multiple_of(c0, 128)
            return y_hbm.at[pl.ds(r, chunk_rows), pl.ds(c, 128)]

        def advance(row0, c0):
            wrap = c0 + 128 >= n_c_sc
            row0n = jnp.where(wrap, row0 + n_hw, row0)
            c0n = jnp.where(wrap, jnp.int32(0), c0 + 128)
            return row0n, c0n

        r00 = row_base + jnp.int32(0)
        c00 = jnp.int32(0)
        pltpu.async_copy(src(r00, c00, 0), buf0, sem0)
        pltpu.async_copy(src(r00, c00, 1), buf1, sem1)

        def unit_body(u, carry):
            row0, c0 = carry
            row0n, c0n = advance(row0, c0)
            ms = [(negv, negv, negv, negv) for _ in range(8)]
            for q in range(n_chunk):
                buf = bufs[q % 2]
                sem = sems[q % 2]
                pltpu.make_async_copy(src(r00, c00, 0), buf, sem).wait()
                for p in range(8):
                    def pass_body(t, m, _p=p, _buf=buf):
                        vs = []
                        for w8 in range(8):
                            vs.append(_buf[t * 8 + w8,
                                           pl.ds(_p * _L, _L)])
                            if len(vs) == 4:
                                m = _absorb4(m, tuple(vs))
                                vs = []
                        return m
                    ms[p] = lax.fori_loop(0, n_tiles, pass_body, ms[p])
                nq = q + 2
                if nq < n_chunk:
                    pltpu.async_copy(src(row0, c0, nq), buf, sem)
                else:
                    @pl.when(u + 1 < n_units)
                    def _():
                        pltpu.async_copy(src(row0n, c0n, nq - n_chunk),
                                         buf, sem)
            obase = u * 128
            for p in range(8):
                m0, m1, m2, m3 = ms[p]
                res = m0 * wr[0] + m1 * wr[1] + m2 * wr[2] + m3 * wr[3]
                outv[pl.ds(obase + p * _L, _L)] = res
            return row0n, c0n

        lax.fori_loop(0, n_units, unit_body, (r00, c00))

        pltpu.sync_copy(outv, out_hbm.at[pl.ds(wid * out_per_w, out_per_w)])

    return pl.kernel(
        body,
        out_type=jax.ShapeDtypeStruct((n_b * n_c_sc,), jnp.float32),
        mesh=mesh,
        compiler_params=pltpu.CompilerParams(needs_layout_passes=False),
        scratch_types=[
            pltpu.VMEM((_K, _L), jnp.float32),
            pltpu.VMEM((chunk_rows, 128), jnp.float32),
            pltpu.VMEM((chunk_rows, 128), jnp.float32),
            pltpu.VMEM((out_per_w,), jnp.float32),
            pltpu.SemaphoreType.DMA,
            pltpu.SemaphoreType.DMA,
        ],
    )


def _tc_block(w_ref, y_ref, o_ref):
    n_hw = y_ref.shape[0]
    negv = jnp.full((8, y_ref.shape[1]), _NEG, jnp.float32)

    n_acc = 4
    rows_per_iter = 32 * n_acc

    def body(i, st):
        out = []
        for a in range(n_acc):
            base = i * rows_per_iter + a * 32
            vs = tuple(y_ref[pl.ds(base + t * 8, 8), :] for t in range(4))
            out.append(_absorb4(st[a], vs))
        return tuple(out)

    neg4 = (negv, negv, negv, negv)
    sts = lax.fori_loop(0, n_hw // rows_per_iter, body, (neg4,) * n_acc)
    while len(sts) > 1:
        sts = tuple(_merge44(sts[2 * i], sts[2 * i + 1])
                    for i in range(len(sts) // 2))
    ms = sts[0]
    for h in (4, 2, 1):
        a = tuple(m[:h] for m in ms)
        b = tuple(m[h:2 * h] for m in ms)
        ms = _merge44(a, b)
    acc = ms[0] * w_ref[0]
    for r in range(1, _K):
        acc = acc + ms[r] * w_ref[r]
    o_ref[...] = acc.reshape(o_ref.shape)


def _make_tc_pool(n_b_tc, n_c, n_hw, b_off):
    grid = (n_b_tc,)
    return pl.pallas_call(
        _tc_block,
        grid=grid,
        in_specs=[
            pl.BlockSpec(memory_space=pltpu.SMEM),
            pl.BlockSpec((n_hw, n_c), lambda i: (b_off + i, 0)),
        ],
        out_specs=pl.BlockSpec((1, 1, n_c), lambda i: (i, 0, 0)),
        out_shape=jax.ShapeDtypeStruct((n_b_tc, 1, n_c), jnp.float32),
        compiler_params=pltpu.CompilerParams(
            dimension_semantics=("arbitrary",)),
    )


_B_SC = 32


def kernel(x, weights):
    b, c, h, w = x.shape
    n_hw = h * w
    assert c % 128 == 0 and b % _NW == 0 and n_hw % 32 == 0
    y = x.transpose(0, 2, 3, 1).reshape(b * n_hw, c)
    wmat = jnp.broadcast_to(
        weights.reshape(_K, 1).astype(jnp.float32) / _K, (_K, _L))
    n_b_sc = _B_SC if 0 < _B_SC < b else b
    pool = _make_pool(n_b_sc, c, c, n_hw)
    out_sc = pool(y, wmat).reshape(n_b_sc, c)
    if n_b_sc < b:
        wvec = weights.reshape(_K).astype(jnp.float32) / _K
        tc_pool = _make_tc_pool(b - n_b_sc, c, n_hw, n_b_sc)
        out_tc = tc_pool(wvec, y).reshape(b - n_b_sc, c)
        out = jnp.concatenate([out_sc, out_tc], axis=0)
    else:
        out = out_sc
    return out.reshape(b, c, 1, 1)
